# Initial kernel scaffold; baseline (speedup 1.0000x reference)
import jax
import jax.numpy as jnp
from jax.experimental import pallas as pl


def kernel(preds, targets, effective_maps):
    raise NotImplementedError("write your pallas kernel here")


# speedup vs baseline: 18.5058x; 18.5058x over previous
import jax
import jax.numpy as jnp
from jax import lax
from jax.experimental import pallas as pl
from jax.experimental.pallas import tpu as pltpu

_OHEM_RATIO = 3.0
_SMOOTH = 1e-06
_ONE_BITS = 0x3F800000


def _loss_body(pt_ref, tt_ref, pk_ref, tk_ref, eff_ref, text_ref, kern_ref):
    logits_t = pt_ref[0]
    tt = tt_ref[0]
    eff = eff_ref[0]
    pred = jax.nn.sigmoid(logits_t)

    pos = tt > 0.5
    neg = jnp.logical_not(pos)
    effg = eff > 0.5

    pos_num = jnp.sum(jnp.where(pos & effg, 1, 0))
    neg_total = jnp.sum(jnp.where(neg, 1, 0))
    neg_num = jnp.minimum(
        pos_num.astype(jnp.float32) * _OHEM_RATIO,
        neg_total.astype(jnp.float32),
    ).astype(jnp.int32)

    bits = lax.bitcast_convert_type(pred, jnp.int32)
    mbits = jnp.where(neg, bits, -1)

    def step(_, lohi):
        lo, hi = lohi
        mid = lo + (hi - lo + 1) // 2
        cnt = jnp.sum(jnp.where(mbits >= mid, 1, 0))
        ok = cnt >= neg_num
        return jnp.where(ok, mid, lo), jnp.where(ok, hi, mid - 1)

    lo, _ = lax.fori_loop(
        0, 30, step, (jnp.int32(0), jnp.int32(_ONE_BITS)), unroll=False
    )

    sel = ((bits >= lo) | pos) & effg
    cond = (pos_num == 0) | (neg_num == 0)
    sel_f = jnp.where(
        cond,
        (eff != 0.0).astype(jnp.float32),
        sel.astype(jnp.float32),
    )

    t_f = pos.astype(jnp.float32) * sel_f
    p_f = pred * sel_f
    pg = jnp.sum(p_f * t_f)
    p2 = jnp.sum(p_f * p_f)
    g2 = jnp.sum(t_f * t_f)
    text_loss = 1.0 - (2.0 * pg + _SMOOTH) / (p2 + g2 + _SMOOTH)

    pred_k = jax.nn.sigmoid(pk_ref[0])
    sel2 = ((pred > 0.5) & effg).astype(jnp.float32)
    tk_f = (tk_ref[0] > 0.5).astype(jnp.float32) * sel2
    pk_f = pred_k * sel2
    pg2 = jnp.sum(pk_f * tk_f)
    p2b = jnp.sum(pk_f * pk_f)
    g2b = jnp.sum(tk_f * tk_f)
    kernel_loss = 1.0 - (2.0 * pg2 + _SMOOTH) / (p2b + g2b + _SMOOTH)

    text_ref[0, 0, :] = jnp.full((128,), text_loss, dtype=jnp.float32)
    kern_ref[0, 0, :] = jnp.full((128,), kernel_loss, dtype=jnp.float32)


def kernel(preds, targets, effective_maps):
    n, _, h, w = preds.shape
    img_spec = pl.BlockSpec((1, h, w), lambda i: (i, 0, 0))
    out_spec = pl.BlockSpec((1, 1, 128), lambda i: (i, 0, 0))
    text, kern = pl.pallas_call(
        _loss_body,
        grid=(n,),
        in_specs=[img_spec] * 5,
        out_specs=[out_spec, out_spec],
        out_shape=[
            jax.ShapeDtypeStruct((n, 1, 128), jnp.float32),
            jax.ShapeDtypeStruct((n, 1, 128), jnp.float32),
        ],
        compiler_params=pltpu.CompilerParams(
            dimension_semantics=("arbitrary",),
        ),
    )(
        preds[:, 0, :, :],
        targets[:, 0, :, :],
        preds[:, 1, :, :],
        targets[:, 1, :, :],
        effective_maps,
    )
    return text[:, 0, 0], kern[:, 0, 0]


# speedup vs baseline: 22.1896x; 1.1991x over previous
import jax
import jax.numpy as jnp
from jax import lax
from jax.experimental import pallas as pl
from jax.experimental.pallas import tpu as pltpu

_OHEM_RATIO = 3.0
_SMOOTH = 1e-06
_ONE_BITS = 0x3F800000


def _loss_body(pt_ref, tt_ref, pk_ref, tk_ref, eff_ref, text_ref, kern_ref):
    logits_t = pt_ref[0, 0]
    tt = tt_ref[0, 0]
    eff = eff_ref[0]
    pred = jax.nn.sigmoid(logits_t)

    pos = tt > 0.5
    neg = jnp.logical_not(pos)
    effg = eff > 0.5

    pos_num = jnp.sum(jnp.where(pos & effg, 1, 0))
    neg_total = jnp.sum(jnp.where(neg, 1, 0))
    neg_num = jnp.minimum(
        pos_num.astype(jnp.float32) * _OHEM_RATIO,
        neg_total.astype(jnp.float32),
    ).astype(jnp.int32)

    bits = lax.bitcast_convert_type(pred, jnp.int32)
    mbits = jnp.where(neg, bits, -1)

    def step(_, lohi):
        lo, hi = lohi
        mid = lo + (hi - lo + 1) // 2
        cnt = jnp.sum(jnp.where(mbits >= mid, 1, 0))
        ok = cnt >= neg_num
        return jnp.where(ok, mid, lo), jnp.where(ok, hi, mid - 1)

    lo, _ = lax.fori_loop(
        0, 30, step, (jnp.int32(0), jnp.int32(_ONE_BITS)), unroll=False
    )

    sel = ((bits >= lo) | pos) & effg
    cond = (pos_num == 0) | (neg_num == 0)
    sel_f = jnp.where(
        cond,
        (eff != 0.0).astype(jnp.float32),
        sel.astype(jnp.float32),
    )

    t_f = pos.astype(jnp.float32) * sel_f
    p_f = pred * sel_f
    pg = jnp.sum(p_f * t_f)
    p2 = jnp.sum(p_f * p_f)
    g2 = jnp.sum(t_f * t_f)
    text_loss = 1.0 - (2.0 * pg + _SMOOTH) / (p2 + g2 + _SMOOTH)

    pred_k = jax.nn.sigmoid(pk_ref[0, 0])
    sel2 = ((pred > 0.5) & effg).astype(jnp.float32)
    tk_f = (tk_ref[0, 0] > 0.5).astype(jnp.float32) * sel2
    pk_f = pred_k * sel2
    pg2 = jnp.sum(pk_f * tk_f)
    p2b = jnp.sum(pk_f * pk_f)
    g2b = jnp.sum(tk_f * tk_f)
    kernel_loss = 1.0 - (2.0 * pg2 + _SMOOTH) / (p2b + g2b + _SMOOTH)

    text_ref[0, 0, :] = jnp.full((128,), text_loss, dtype=jnp.float32)
    kern_ref[0, 0, :] = jnp.full((128,), kernel_loss, dtype=jnp.float32)


def kernel(preds, targets, effective_maps):
    n, _, h, w = preds.shape
    img_spec = pl.BlockSpec((1, h, w), lambda i: (i, 0, 0))
    ch0_spec = pl.BlockSpec((1, 1, h, w), lambda i: (i, 0, 0, 0))
    ch1_spec = pl.BlockSpec((1, 1, h, w), lambda i: (i, 1, 0, 0))
    out_spec = pl.BlockSpec((1, 1, 128), lambda i: (i, 0, 0))
    text, kern = pl.pallas_call(
        _loss_body,
        grid=(n,),
        in_specs=[ch0_spec, ch0_spec, ch1_spec, ch1_spec, img_spec],
        out_specs=[out_spec, out_spec],
        out_shape=[
            jax.ShapeDtypeStruct((n, 1, 128), jnp.float32),
            jax.ShapeDtypeStruct((n, 1, 128), jnp.float32),
        ],
        compiler_params=pltpu.CompilerParams(
            dimension_semantics=("arbitrary",),
        ),
    )(preds, targets, preds, targets, effective_maps)
    return text[:, 0, 0], kern[:, 0, 0]


# speedup vs baseline: 30.2736x; 1.3643x over previous
import jax
import jax.numpy as jnp
from jax import lax
from jax.experimental import pallas as pl
from jax.experimental.pallas import tpu as pltpu

_OHEM_RATIO = 3.0
_SMOOTH = 1e-06
_ONE_BITS = 0x3F800000


def _loss_body(pt_ref, tt_ref, pk_ref, tk_ref, eff_ref, text_ref, kern_ref):
    logits_t = pt_ref[0, 0]
    tt = tt_ref[0, 0]
    eff = eff_ref[0]
    pred = jax.nn.sigmoid(logits_t)

    pos = tt > 0.5
    neg = jnp.logical_not(pos)
    effg = eff > 0.5

    pos_num = jnp.sum(
        jnp.where(pos & effg, 1, 0), axis=(0, 1), keepdims=True
    )
    neg_total = jnp.sum(jnp.where(neg, 1, 0), axis=(0, 1), keepdims=True)
    neg_num = jnp.minimum(
        pos_num.astype(jnp.float32) * _OHEM_RATIO,
        neg_total.astype(jnp.float32),
    ).astype(jnp.int32)

    bits = lax.bitcast_convert_type(pred, jnp.int32)
    mbits = jnp.where(neg, bits, -1)

    def step(_, lohi):
        lo, hi = lohi
        s = (hi - lo + 4) // 4
        m1 = lo + s
        m2 = lo + 2 * s
        m3 = lo + 3 * s
        c1 = jnp.sum((mbits >= m1).astype(jnp.int32), axis=(0, 1), keepdims=True)
        c2 = jnp.sum((mbits >= m2).astype(jnp.int32), axis=(0, 1), keepdims=True)
        c3 = jnp.sum((mbits >= m3).astype(jnp.int32), axis=(0, 1), keepdims=True)
        ok1 = c1 >= neg_num
        ok2 = c2 >= neg_num
        ok3 = c3 >= neg_num
        lo2 = jnp.where(ok1, m1, lo)
        lo2 = jnp.where(ok2, m2, lo2)
        lo2 = jnp.where(ok3, m3, lo2)
        hi2 = jnp.where(jnp.logical_not(ok3), m3 - 1, hi)
        hi2 = jnp.where(jnp.logical_not(ok2), m2 - 1, hi2)
        hi2 = jnp.where(jnp.logical_not(ok1), m1 - 1, hi2)
        return lo2, hi2

    lo, _ = lax.fori_loop(
        0, 15, step,
        (jnp.zeros((1, 1), jnp.int32), jnp.full((1, 1), _ONE_BITS, jnp.int32)),
        unroll=False,
    )

    sel = ((bits >= lo) | pos) & effg
    cond = (pos_num == 0) | (neg_num == 0)
    sel_f = jnp.where(
        cond,
        (eff != 0.0).astype(jnp.float32),
        sel.astype(jnp.float32),
    )

    t_f = pos.astype(jnp.float32) * sel_f
    p_f = pred * sel_f
    pg = jnp.sum(p_f * t_f)
    p2 = jnp.sum(p_f * p_f)
    g2 = jnp.sum(t_f * t_f)
    text_loss = 1.0 - (2.0 * pg + _SMOOTH) / (p2 + g2 + _SMOOTH)

    pred_k = jax.nn.sigmoid(pk_ref[0, 0])
    sel2 = ((pred > 0.5) & effg).astype(jnp.float32)
    tk_f = (tk_ref[0, 0] > 0.5).astype(jnp.float32) * sel2
    pk_f = pred_k * sel2
    pg2 = jnp.sum(pk_f * tk_f)
    p2b = jnp.sum(pk_f * pk_f)
    g2b = jnp.sum(tk_f * tk_f)
    kernel_loss = 1.0 - (2.0 * pg2 + _SMOOTH) / (p2b + g2b + _SMOOTH)

    text_ref[0, 0, :] = jnp.full((128,), text_loss, dtype=jnp.float32)
    kern_ref[0, 0, :] = jnp.full((128,), kernel_loss, dtype=jnp.float32)


def kernel(preds, targets, effective_maps):
    n, _, h, w = preds.shape
    img_spec = pl.BlockSpec((1, h, w), lambda i: (i, 0, 0))
    ch0_spec = pl.BlockSpec((1, 1, h, w), lambda i: (i, 0, 0, 0))
    ch1_spec = pl.BlockSpec((1, 1, h, w), lambda i: (i, 1, 0, 0))
    out_spec = pl.BlockSpec((1, 1, 128), lambda i: (i, 0, 0))
    text, kern = pl.pallas_call(
        _loss_body,
        grid=(n,),
        in_specs=[ch0_spec, ch0_spec, ch1_spec, ch1_spec, img_spec],
        out_specs=[out_spec, out_spec],
        out_shape=[
            jax.ShapeDtypeStruct((n, 1, 128), jnp.float32),
            jax.ShapeDtypeStruct((n, 1, 128), jnp.float32),
        ],
        compiler_params=pltpu.CompilerParams(
            dimension_semantics=("arbitrary",),
        ),
    )(preds, targets, preds, targets, effective_maps)
    return text[:, 0, 0], kern[:, 0, 0]


# speedup vs baseline: 33.4289x; 1.1042x over previous
import jax
import jax.numpy as jnp
from jax import lax
from jax.experimental import pallas as pl
from jax.experimental.pallas import tpu as pltpu

_OHEM_RATIO = 3.0
_SMOOTH = 1e-06
_ONE_BITS = 0x3F800000


def _loss_body(pt_ref, tt_ref, pk_ref, tk_ref, eff_ref, text_ref, kern_ref,
               mb_ref):
    logits_t = pt_ref[0, 0]
    tt = tt_ref[0, 0]
    eff = eff_ref[0]
    pred = jax.nn.sigmoid(logits_t)

    pos = tt > 0.5
    neg = jnp.logical_not(pos)
    effg = eff > 0.5

    pos_num = jnp.sum(
        jnp.where(pos & effg, 1, 0), axis=(0, 1), keepdims=True
    )
    neg_total = jnp.sum(jnp.where(neg, 1, 0), axis=(0, 1), keepdims=True)
    neg_num = jnp.minimum(
        pos_num.astype(jnp.float32) * _OHEM_RATIO,
        neg_total.astype(jnp.float32),
    ).astype(jnp.int32)

    bits = lax.bitcast_convert_type(pred, jnp.int32)
    mbits = jnp.where(neg, bits, -1)

    mb_ref[...] = mbits

    def step(_, lohi):
        lo, hi = lohi
        s = (hi - lo + 4) // 4
        m1 = lo + s
        m2 = lo + 2 * s
        m3 = lo + 3 * s
        acc1 = jnp.zeros((8, 512), jnp.int32)
        acc2 = jnp.zeros((8, 512), jnp.int32)
        acc3 = jnp.zeros((8, 512), jnp.int32)
        for r in range(64):
            chunk = mb_ref[pl.ds(8 * r, 8), :]
            acc1 = acc1 + (chunk >= m1).astype(jnp.int32)
            acc2 = acc2 + (chunk >= m2).astype(jnp.int32)
            acc3 = acc3 + (chunk >= m3).astype(jnp.int32)
        c1 = jnp.sum(acc1, axis=(0, 1), keepdims=True)
        c2 = jnp.sum(acc2, axis=(0, 1), keepdims=True)
        c3 = jnp.sum(acc3, axis=(0, 1), keepdims=True)
        ok1 = c1 >= neg_num
        ok2 = c2 >= neg_num
        ok3 = c3 >= neg_num
        lo2 = jnp.where(ok1, m1, lo)
        lo2 = jnp.where(ok2, m2, lo2)
        lo2 = jnp.where(ok3, m3, lo2)
        hi2 = jnp.where(jnp.logical_not(ok3), m3 - 1, hi)
        hi2 = jnp.where(jnp.logical_not(ok2), m2 - 1, hi2)
        hi2 = jnp.where(jnp.logical_not(ok1), m1 - 1, hi2)
        return lo2, hi2

    lo, _ = lax.fori_loop(
        0, 15, step,
        (jnp.zeros((1, 1), jnp.int32), jnp.full((1, 1), _ONE_BITS, jnp.int32)),
        unroll=False,
    )

    sel = ((bits >= lo) | pos) & effg
    cond = (pos_num == 0) | (neg_num == 0)
    sel_f = jnp.where(
        cond,
        (eff != 0.0).astype(jnp.float32),
        sel.astype(jnp.float32),
    )

    t_f = pos.astype(jnp.float32) * sel_f
    p_f = pred * sel_f
    pg = jnp.sum(p_f * t_f)
    p2 = jnp.sum(p_f * p_f)
    g2 = jnp.sum(t_f * t_f)
    text_loss = 1.0 - (2.0 * pg + _SMOOTH) / (p2 + g2 + _SMOOTH)

    pred_k = jax.nn.sigmoid(pk_ref[0, 0])
    sel2 = ((pred > 0.5) & effg).astype(jnp.float32)
    tk_f = (tk_ref[0, 0] > 0.5).astype(jnp.float32) * sel2
    pk_f = pred_k * sel2
    pg2 = jnp.sum(pk_f * tk_f)
    p2b = jnp.sum(pk_f * pk_f)
    g2b = jnp.sum(tk_f * tk_f)
    kernel_loss = 1.0 - (2.0 * pg2 + _SMOOTH) / (p2b + g2b + _SMOOTH)

    text_ref[0, 0, :] = jnp.full((128,), text_loss, dtype=jnp.float32)
    kern_ref[0, 0, :] = jnp.full((128,), kernel_loss, dtype=jnp.float32)


def kernel(preds, targets, effective_maps):
    n, _, h, w = preds.shape
    img_spec = pl.BlockSpec((1, h, w), lambda i: (i, 0, 0))
    ch0_spec = pl.BlockSpec((1, 1, h, w), lambda i: (i, 0, 0, 0))
    ch1_spec = pl.BlockSpec((1, 1, h, w), lambda i: (i, 1, 0, 0))
    out_spec = pl.BlockSpec((1, 1, 128), lambda i: (i, 0, 0))
    text, kern = pl.pallas_call(
        _loss_body,
        grid=(n,),
        in_specs=[ch0_spec, ch0_spec, ch1_spec, ch1_spec, img_spec],
        out_specs=[out_spec, out_spec],
        out_shape=[
            jax.ShapeDtypeStruct((n, 1, 128), jnp.float32),
            jax.ShapeDtypeStruct((n, 1, 128), jnp.float32),
        ],
        scratch_shapes=[pltpu.VMEM((512, 512), jnp.int32)],
        compiler_params=pltpu.CompilerParams(
            dimension_semantics=("arbitrary",),
        ),
    )(preds, targets, preds, targets, effective_maps)
    return text[:, 0, 0], kern[:, 0, 0]
